# baseline (speedup 1.0000x reference)
import functools

import jax
import jax.numpy as jnp
from jax import lax
from jax.experimental import pallas as pl
from jax.experimental.pallas import tpu as pltpu
from jax.experimental.pallas import tpu_sc as plsc

N = 10000
E = 320000
D = 128
G = 16

NP = 10240
NT = 10064
EP = 327680
NC = 2
NS = 16
NW = NC * NS
EW = EP // NW
B = 32
GB = 16
NG = EW // (B * GB)
RPW = NP // NS
DR = NP // D
PAD_DST = NT - 1

_f32 = jnp.float32
_i32 = jnp.int32
_bf16 = jnp.bfloat16

import numpy as _np
_SIG = _np.concatenate(
    [_np.concatenate([_np.arange(c * 32, c * 32 + 32, 2),
                      _np.arange(c * 32 + 1, c * 32 + 32, 2)])
     for c in range(4)])
_INV_SIG = _np.argsort(_SIG)


def _lr(v, slope):
    return jnp.maximum(v, slope * v)


def _tc_common(h, asv, adv, h_ref, as_ref, ad_ref, amax_ref):
    h_ref[...] = h.astype(_bf16)
    a_s = jnp.sum(h * asv[None, :], axis=-1)
    a_d = jnp.sum(h * adv[None, :], axis=-1)
    iota = lax.broadcasted_iota(_i32, (NP,), 0)
    amax = jnp.max(jnp.where(iota < N, a_s, -1e30))
    as_ref[...] = a_s
    ad_ref[...] = a_d
    amax_ref[...] = jnp.full((16,), amax, _f32)


def _tc_first_body(x_ref, w_ref, asv_ref, adv_ref,
                   h_ref, as_ref, ad_ref, amax_ref):
    h = jnp.dot(x_ref[...], w_ref[...], preferred_element_type=_f32)
    _tc_common(h, asv_ref[...], adv_ref[...], h_ref, as_ref, ad_ref, amax_ref)


def _tc_next_body(o_ref, d_ref, b_ref, w_ref, asv_ref, adv_ref,
                  h_ref, as_ref, ad_ref, amax_ref):
    den = d_ref[0, :] + d_ref[1, :] + 1e-16
    xin = (o_ref[0] + o_ref[1]) / den[:, None] + b_ref[...][None, :]
    xin = _lr(xin, 0.01)
    h = jnp.dot(xin, w_ref[...], preferred_element_type=_f32)
    _tc_common(h, asv_ref[...], adv_ref[...], h_ref, as_ref, ad_ref, amax_ref)


_TC_OUT = (
    jax.ShapeDtypeStruct((NP, D), _bf16),
    jax.ShapeDtypeStruct((NP,), _f32),
    jax.ShapeDtypeStruct((NP,), _f32),
    jax.ShapeDtypeStruct((16,), _f32),
)


def _tc_first(x_pad, W, a_src, a_dst):
    return pl.pallas_call(_tc_first_body, out_shape=_TC_OUT)(
        x_pad, W, a_src, a_dst)


def _tc_next(o, d, b, W, a_src, a_dst):
    return pl.pallas_call(_tc_next_body, out_shape=_TC_OUT)(
        o, d, b, W, a_src, a_dst)


def _pool_body(o_ref, d_ref, b_ref, batch_ref, out_ref):
    den = d_ref[0, :] + d_ref[1, :] + 1e-16
    h = (o_ref[0] + o_ref[1]) / den[:, None] + b_ref[...][None, :]
    h = h[0:N]
    batch = batch_ref[...]
    seg = lax.broadcasted_iota(_i32, (G, N), 0)
    onehot = (seg == batch[None, :]).astype(_f32)
    sums = jnp.dot(onehot, h, preferred_element_type=_f32)
    counts = jnp.sum(onehot, axis=1)
    out_ref[...] = sums / jnp.maximum(counts, 1.0)[:, None]


def _pool(o, d, b, batch):
    return pl.pallas_call(
        _pool_body,
        out_shape=jax.ShapeDtypeStruct((G, D), _f32),
    )(o, d, b, batch)


_MESH = plsc.VectorSubcoreMesh(
    core_axis_name="c", subcore_axis_name="s", num_cores=NC, num_subcores=NS)


@functools.partial(
    pl.kernel,
    out_type=(
        jax.ShapeDtypeStruct((NC, NP, D), _f32),
        jax.ShapeDtypeStruct((NC, DR, D), _f32),
    ),
    mesh=_MESH,
    compiler_params=pltpu.CompilerParams(
        needs_layout_passes=False, use_tc_tiling_on_sc=False),
    scratch_types=[
        pltpu.VMEM((NT,), _f32),
        pltpu.VMEM((NT,), _f32),
        pltpu.VMEM((DR, D), _f32),
        pltpu.VMEM((16,), _f32),
        pltpu.VMEM((GB, B), _i32),
        pltpu.VMEM((GB, B), _i32),
        pltpu.VMEM((GB * B,), _f32),
        pltpu.VMEM((B, D), _bf16),
        pltpu.VMEM((B, D), _bf16),
        pltpu.VMEM((B, D), _f32),
        pltpu.VMEM((B, D), _f32),
        pltpu.VMEM((DR,), _i32),
        pltpu.VMEM_SHARED((NP, D), _f32),
        pltpu.VMEM_SHARED((DR, D), _f32),
        pltpu.SemaphoreType.DMA,
        pltpu.SemaphoreType.DMA,
        pltpu.SemaphoreType.DMA,
        pltpu.SemaphoreType.DMA,
    ],
)
def _edge_kernel(as_hbm, ad_hbm, am_hbm, src_hbm, dst_hbm, h_hbm,
                 acc_out, den_out,
                 as_v, ad_v, d_v, am_v, sidx, didx, pbuf, rbf0, rbf1,
                 rows0, rows1, id_v, acc_sh, den_sh, semg0, semg1, sems0,
                 sems1):
    cid = lax.axis_index("c")
    sid = lax.axis_index("s")
    wid = cid * NS + sid
    gbase0 = wid * (EW // B)

    pltpu.sync_copy(as_hbm.at[pl.ds(0, NT)], as_v)
    pltpu.sync_copy(ad_hbm.at[pl.ds(0, NT)], ad_v)
    pltpu.sync_copy(am_hbm, am_v)

    zero16 = jnp.zeros((16,), _f32)
    amax = am_v[...]

    def _zero_d(i, carry):
        d_v[i // 8, pl.ds((i % 8) * 16, 16)] = zero16
        return carry

    lax.fori_loop(0, DR * D // 16, _zero_d, 0)

    def _zero_rows(i, carry):
        rows0[i // 8, pl.ds((i % 8) * 16, 16)] = zero16
        return carry

    lax.fori_loop(0, B * D // 16, _zero_rows, 0)

    def _fill_id(i, carry):
        id_v[pl.ds(i * 16, 16)] = lax.iota(_i32, 16) + i * 16
        return carry

    lax.fori_loop(0, DR // 16, _fill_id, 0)

    def _zero_acc(t, carry):
        pltpu.sync_copy(rows0, acc_sh.at[pl.ds(sid * RPW + t * B, B)])
        return carry

    lax.fori_loop(0, RPW // B, _zero_acc, 0)
    pltpu.sync_copy(rows0.at[pl.ds(0, DR // NS)],
                    den_sh.at[pl.ds(sid * (DR // NS), DR // NS)])
    plsc.subcore_barrier()

    rows = (rows0, rows1)
    rbf = (rbf0, rbf1)
    semg = (semg0, semg1)
    sems = (sems0, sems1)

    def _group(g, carry):
        grow = gbase0 + g * GB
        pltpu.sync_copy(src_hbm.at[pl.ds(grow, GB)], sidx)
        pltpu.sync_copy(dst_hbm.at[pl.ds(grow, GB)], didx)

        gat = [None, None]
        gat[0] = pltpu.async_copy(h_hbm.at[sidx.at[0]], rbf0, semg0)

        for jj in range(GB):
            for k in range(B // 16):
                bsl = pl.ds(k * 16, 16)
                s16 = sidx[jj, bsl]
                d16 = didx[jj, bsl]
                sv = plsc.load_gather(as_v, [s16])
                dv = plsc.load_gather(ad_v, [d16])
                z = sv + dv
                e = jnp.maximum(z, 0.2 * z)
                zc = amax + dv
                cg = jnp.maximum(zc, 0.2 * zc)
                p16 = jnp.exp(e - cg)
                p16 = jnp.where(d16 < N, p16, 0.0)
                pbuf[pl.ds(jj * B + k * 16, 16)] = p16
                plsc.addupdate_scatter(d_v, [d16 >> 7, d16 & 127], p16)

        sca = [None, None]
        for jj in range(GB):
            bb = jj & 1
            if jj + 1 < GB:
                gat[1 - bb] = pltpu.async_copy(
                    h_hbm.at[sidx.at[jj + 1]], rbf[1 - bb], semg[1 - bb])
            gat[bb].wait()
            if sca[bb] is not None:
                sca[bb].wait()

            def _scale(r, c2, _jj=jj, _bb=bb):
                a16 = plsc.load_gather(
                    pbuf, [jnp.full((16,), _jj * B, _i32) + r])
                src_b = rbf[_bb]
                dst_f = rows[_bb]
                for c in range(D // 32):
                    v = plsc.bitcast(src_b[r, pl.ds(c * 32, 32)], _i32)
                    lo = plsc.bitcast(v << 16, _f32)
                    hi = plsc.bitcast(v & jnp.int32(-65536), _f32)
                    dst_f[r, pl.ds(c * 32, 16)] = lo * a16
                    dst_f[r, pl.ds(c * 32 + 16, 16)] = hi * a16
                return c2

            lax.fori_loop(0, B, _scale, 0)

            sca[bb] = pltpu.async_copy(
                rows[bb], acc_sh.at[didx.at[jj]], sems[bb], add=True)

        sca[0].wait()
        sca[1].wait()
        return carry

    lax.fori_loop(0, NG, _group, 0)

    pltpu.sync_copy(d_v, den_sh.at[id_v], add=True)
    plsc.subcore_barrier()

    pltpu.sync_copy(den_sh.at[pl.ds(sid * (DR // NS), DR // NS)],
                    den_out.at[cid, pl.ds(sid * (DR // NS), DR // NS)])
    pltpu.sync_copy(acc_sh.at[pl.ds(sid * RPW, RPW)],
                    acc_out.at[cid, pl.ds(sid * RPW, RPW)])


def kernel(x, edge_index, batch, W1, a_src1, a_dst1, b1,
           W2, a_src2, a_dst2, b2, W3, a_src3, a_dst3, b3):
    src = edge_index[0].astype(_i32)
    dst = edge_index[1].astype(_i32)
    ppw = (EP - E) // NW
    pad_src = jnp.arange(NW * ppw, dtype=_i32).reshape(NW, ppw) % N
    pad_dst = N + (jnp.arange(NW * ppw, dtype=_i32).reshape(NW, ppw) % (NT - N))
    srcp = jnp.concatenate(
        [src.reshape(NW, E // NW), pad_src], axis=1).reshape(EP // B, B)
    dstp = jnp.concatenate(
        [dst.reshape(NW, E // NW), pad_dst], axis=1).reshape(EP // B, B)
    x_pad = jnp.concatenate([x, jnp.zeros((NP - N, D), _f32)])
    batch32 = batch.astype(_i32)

    sig = jnp.asarray(_SIG, _i32)
    inv_sig = jnp.asarray(_INV_SIG, _i32)

    h, a_s, a_d, am = _tc_first(x_pad, W1, a_src1, a_dst1)
    o, d = _edge_kernel(a_s, a_d, am, srcp, dstp, h)
    d = d.reshape(NC, NP)

    h, a_s, a_d, am = _tc_next(o, d, b1[sig], W2[sig, :], a_src2, a_dst2)
    o, d = _edge_kernel(a_s, a_d, am, srcp, dstp, h)
    d = d.reshape(NC, NP)

    h, a_s, a_d, am = _tc_next(o, d, b2[sig], W3[sig, :], a_src3, a_dst3)
    o, d = _edge_kernel(a_s, a_d, am, srcp, dstp, h)
    d = d.reshape(NC, NP)

    pooled = _pool(o, d, b3[sig], batch32)
    return pooled[:, inv_sig]


# speedup vs baseline: 1.0011x; 1.0011x over previous
import functools

import jax
import jax.numpy as jnp
from jax import lax
from jax.experimental import pallas as pl
from jax.experimental.pallas import tpu as pltpu
from jax.experimental.pallas import tpu_sc as plsc

N = 10000
E = 320000
D = 128
G = 16

NP = 10240
NT = 10064
EP = 327680
NC = 2
NS = 16
NW = NC * NS
EW = EP // NW
B = 64
HB = 32
GB = 8
NG = EW // (B * GB)
RPW = NP // NS
DR = NP // D
PAD_DST = NT - 1

_f32 = jnp.float32
_i32 = jnp.int32
_bf16 = jnp.bfloat16

import numpy as _np
_SIG = _np.concatenate(
    [_np.concatenate([_np.arange(c * 32, c * 32 + 32, 2),
                      _np.arange(c * 32 + 1, c * 32 + 32, 2)])
     for c in range(4)])
_INV_SIG = _np.argsort(_SIG)


def _lr(v, slope):
    return jnp.maximum(v, slope * v)


def _tc_common(h, asv, adv, h_ref, as_ref, ad_ref, amax_ref):
    h_ref[...] = h.astype(_bf16)
    a_s = jnp.sum(h * asv[None, :], axis=-1)
    a_d = jnp.sum(h * adv[None, :], axis=-1)
    iota = lax.broadcasted_iota(_i32, (NP,), 0)
    amax = jnp.max(jnp.where(iota < N, a_s, -1e30))
    as_ref[...] = a_s
    ad_ref[...] = a_d
    amax_ref[...] = jnp.full((16,), amax, _f32)


def _tc_first_body(x_ref, w_ref, asv_ref, adv_ref,
                   h_ref, as_ref, ad_ref, amax_ref):
    h = jnp.dot(x_ref[...], w_ref[...], preferred_element_type=_f32)
    _tc_common(h, asv_ref[...], adv_ref[...], h_ref, as_ref, ad_ref, amax_ref)


def _tc_next_body(o_ref, d_ref, b_ref, w_ref, asv_ref, adv_ref,
                  h_ref, as_ref, ad_ref, amax_ref):
    den = d_ref[0, :] + d_ref[1, :] + 1e-16
    xin = (o_ref[0] + o_ref[1]) / den[:, None] + b_ref[...][None, :]
    xin = _lr(xin, 0.01)
    h = jnp.dot(xin, w_ref[...], preferred_element_type=_f32)
    _tc_common(h, asv_ref[...], adv_ref[...], h_ref, as_ref, ad_ref, amax_ref)


_TC_OUT = (
    jax.ShapeDtypeStruct((NP, D), _bf16),
    jax.ShapeDtypeStruct((NP,), _f32),
    jax.ShapeDtypeStruct((NP,), _f32),
    jax.ShapeDtypeStruct((16,), _f32),
)


def _tc_first(x_pad, W, a_src, a_dst):
    return pl.pallas_call(_tc_first_body, out_shape=_TC_OUT)(
        x_pad, W, a_src, a_dst)


def _tc_next(o, d, b, W, a_src, a_dst):
    return pl.pallas_call(_tc_next_body, out_shape=_TC_OUT)(
        o, d, b, W, a_src, a_dst)


def _pool_body(o_ref, d_ref, b_ref, batch_ref, out_ref):
    den = d_ref[0, :] + d_ref[1, :] + 1e-16
    h = (o_ref[0] + o_ref[1]) / den[:, None] + b_ref[...][None, :]
    h = h[0:N]
    batch = batch_ref[...]
    seg = lax.broadcasted_iota(_i32, (G, N), 0)
    onehot = (seg == batch[None, :]).astype(_f32)
    sums = jnp.dot(onehot, h, preferred_element_type=_f32)
    counts = jnp.sum(onehot, axis=1)
    out_ref[...] = sums / jnp.maximum(counts, 1.0)[:, None]


def _pool(o, d, b, batch):
    return pl.pallas_call(
        _pool_body,
        out_shape=jax.ShapeDtypeStruct((G, D), _f32),
    )(o, d, b, batch)


_MESH = plsc.VectorSubcoreMesh(
    core_axis_name="c", subcore_axis_name="s", num_cores=NC, num_subcores=NS)


@functools.partial(
    pl.kernel,
    out_type=(
        jax.ShapeDtypeStruct((NC, NP, D), _f32),
        jax.ShapeDtypeStruct((NC, DR, D), _f32),
    ),
    mesh=_MESH,
    compiler_params=pltpu.CompilerParams(
        needs_layout_passes=False, use_tc_tiling_on_sc=False),
    scratch_types=[
        pltpu.VMEM((NT,), _f32),
        pltpu.VMEM((NT,), _f32),
        pltpu.VMEM((DR, D), _f32),
        pltpu.VMEM((16,), _f32),
        pltpu.VMEM((GB, B), _i32),
        pltpu.VMEM((2 * GB, HB), _i32),
        pltpu.VMEM((GB * B,), _f32),
        pltpu.VMEM((B, D), _bf16),
        pltpu.VMEM((B, D), _bf16),
        pltpu.VMEM((HB, D), _f32),
        pltpu.VMEM((HB, D), _f32),
        pltpu.VMEM((DR,), _i32),
        pltpu.VMEM_SHARED((NP, D), _f32),
        pltpu.VMEM_SHARED((DR, D), _f32),
        pltpu.SemaphoreType.DMA,
        pltpu.SemaphoreType.DMA,
        pltpu.SemaphoreType.DMA,
        pltpu.SemaphoreType.DMA,
    ],
)
def _edge_kernel(as_hbm, ad_hbm, am_hbm, src_hbm, dst_hbm, h_hbm,
                 acc_out, den_out,
                 as_v, ad_v, d_v, am_v, sidx, didx, pbuf, rbf0, rbf1,
                 rows0, rows1, id_v, acc_sh, den_sh, semg0, semg1, sems0,
                 sems1):
    cid = lax.axis_index("c")
    sid = lax.axis_index("s")
    wid = cid * NS + sid
    gbase0 = wid * (EW // B)

    pltpu.sync_copy(as_hbm.at[pl.ds(0, NT)], as_v)
    pltpu.sync_copy(ad_hbm.at[pl.ds(0, NT)], ad_v)
    pltpu.sync_copy(am_hbm, am_v)

    zero16 = jnp.zeros((16,), _f32)
    amax = am_v[...]

    def _zero_d(i, carry):
        d_v[i // 8, pl.ds((i % 8) * 16, 16)] = zero16
        return carry

    lax.fori_loop(0, DR * D // 16, _zero_d, 0)

    def _zero_rows(i, carry):
        rows0[i // 8, pl.ds((i % 8) * 16, 16)] = zero16
        return carry

    lax.fori_loop(0, HB * D // 16, _zero_rows, 0)

    def _fill_id(i, carry):
        id_v[pl.ds(i * 16, 16)] = lax.iota(_i32, 16) + i * 16
        return carry

    lax.fori_loop(0, DR // 16, _fill_id, 0)

    def _zero_acc(t, carry):
        pltpu.sync_copy(rows0, acc_sh.at[pl.ds(sid * RPW + t * HB, HB)])
        return carry

    lax.fori_loop(0, RPW // HB, _zero_acc, 0)
    pltpu.sync_copy(rows0.at[pl.ds(0, DR // NS)],
                    den_sh.at[pl.ds(sid * (DR // NS), DR // NS)])
    plsc.subcore_barrier()

    rows = (rows0, rows1)
    rbf = (rbf0, rbf1)
    semg = (semg0, semg1)
    sems = (sems0, sems1)

    def _group(g, carry):
        grow = gbase0 + g * GB
        pltpu.sync_copy(src_hbm.at[pl.ds(grow, GB)], sidx)
        pltpu.sync_copy(dst_hbm.at[pl.ds(2 * grow, 2 * GB)], didx)

        gat = [None, None]
        gat[0] = pltpu.async_copy(h_hbm.at[sidx.at[0]], rbf0, semg0)

        for jj in range(GB):
            for k in range(B // 16):
                bsl = pl.ds(k * 16, 16)
                s16 = sidx[jj, bsl]
                d16 = didx[2 * jj + k // 2, pl.ds((k % 2) * 16, 16)]
                sv = plsc.load_gather(as_v, [s16])
                dv = plsc.load_gather(ad_v, [d16])
                z = sv + dv
                e = jnp.maximum(z, 0.2 * z)
                zc = amax + dv
                cg = jnp.maximum(zc, 0.2 * zc)
                p16 = jnp.exp(e - cg)
                p16 = jnp.where(d16 < N, p16, 0.0)
                pbuf[pl.ds(jj * B + k * 16, 16)] = p16
                plsc.addupdate_scatter(d_v, [d16 >> 7, d16 & 127], p16)

        sca = [None, None]
        for jj in range(GB):
            bb = jj & 1
            if jj + 1 < GB:
                gat[1 - bb] = pltpu.async_copy(
                    h_hbm.at[sidx.at[jj + 1]], rbf[1 - bb], semg[1 - bb])
            gat[bb].wait()

            for half in range(2):
                if sca[half] is not None:
                    sca[half].wait()

                def _scale(r, c2, _jj=jj, _bb=bb, _hf=half):
                    a16 = plsc.load_gather(
                        pbuf, [jnp.full((16,), _jj * B + _hf * HB, _i32) + r])
                    src_b = rbf[_bb]
                    dst_f = rows[_hf]
                    for c in range(D // 32):
                        v = plsc.bitcast(
                            src_b[_hf * HB + r, pl.ds(c * 32, 32)], _i32)
                        lo = plsc.bitcast(v << 16, _f32)
                        hi = plsc.bitcast(v & jnp.int32(-65536), _f32)
                        dst_f[r, pl.ds(c * 32, 16)] = lo * a16
                        dst_f[r, pl.ds(c * 32 + 16, 16)] = hi * a16
                    return c2

                lax.fori_loop(0, HB, _scale, 0)

                sca[half] = pltpu.async_copy(
                    rows[half], acc_sh.at[didx.at[2 * jj + half]],
                    sems[half], add=True)

        sca[0].wait()
        sca[1].wait()
        return carry

    lax.fori_loop(0, NG, _group, 0)

    pltpu.sync_copy(d_v, den_sh.at[id_v], add=True)
    plsc.subcore_barrier()

    pltpu.sync_copy(den_sh.at[pl.ds(sid * (DR // NS), DR // NS)],
                    den_out.at[cid, pl.ds(sid * (DR // NS), DR // NS)])
    pltpu.sync_copy(acc_sh.at[pl.ds(sid * RPW, RPW)],
                    acc_out.at[cid, pl.ds(sid * RPW, RPW)])


def kernel(x, edge_index, batch, W1, a_src1, a_dst1, b1,
           W2, a_src2, a_dst2, b2, W3, a_src3, a_dst3, b3):
    src = edge_index[0].astype(_i32)
    dst = edge_index[1].astype(_i32)
    ppw = (EP - E) // NW
    pad_src = jnp.arange(NW * ppw, dtype=_i32).reshape(NW, ppw) % N
    pad_dst = N + (jnp.arange(NW * ppw, dtype=_i32).reshape(NW, ppw) % (NT - N))
    srcp = jnp.concatenate(
        [src.reshape(NW, E // NW), pad_src], axis=1).reshape(EP // B, B)
    dstp = jnp.concatenate(
        [dst.reshape(NW, E // NW), pad_dst], axis=1).reshape(EP // HB, HB)
    x_pad = jnp.concatenate([x, jnp.zeros((NP - N, D), _f32)])
    batch32 = batch.astype(_i32)

    sig = jnp.asarray(_SIG, _i32)
    inv_sig = jnp.asarray(_INV_SIG, _i32)

    h, a_s, a_d, am = _tc_first(x_pad, W1, a_src1, a_dst1)
    o, d = _edge_kernel(a_s, a_d, am, srcp, dstp, h)
    d = d.reshape(NC, NP)

    h, a_s, a_d, am = _tc_next(o, d, b1[sig], W2[sig, :], a_src2, a_dst2)
    o, d = _edge_kernel(a_s, a_d, am, srcp, dstp, h)
    d = d.reshape(NC, NP)

    h, a_s, a_d, am = _tc_next(o, d, b2[sig], W3[sig, :], a_src3, a_dst3)
    o, d = _edge_kernel(a_s, a_d, am, srcp, dstp, h)
    d = d.reshape(NC, NP)

    pooled = _pool(o, d, b3[sig], batch32)
    return pooled[:, inv_sig]


# speedup vs baseline: 1.5742x; 1.5724x over previous
import functools

import jax
import jax.numpy as jnp
from jax import lax
from jax.experimental import pallas as pl
from jax.experimental.pallas import tpu as pltpu
from jax.experimental.pallas import tpu_sc as plsc

N = 10000
E = 320000
D = 128
G = 16

NP = 10240
NT = 10064
EP = 327680
NC = 2
NS = 16
NW = NC * NS
EW = EP // NW
B = 64
GB = 8
NG = EW // (B * GB)
RPW = NP // NS
DR = NP // D
PAD_DST = NT - 1

_f32 = jnp.float32
_i32 = jnp.int32


def _lr(v, slope):
    return jnp.maximum(v, slope * v)


def _tc_common(h, asv, adv, h_ref, as_ref, ad_ref, amax_ref):
    h_ref[...] = h
    a_s = jnp.sum(h * asv[None, :], axis=-1)
    a_d = jnp.sum(h * adv[None, :], axis=-1)
    iota = lax.broadcasted_iota(_i32, (NP,), 0)
    amax = jnp.max(jnp.where(iota < N, a_s, -1e30))
    as_ref[...] = a_s
    ad_ref[...] = a_d
    amax_ref[...] = jnp.full((16,), amax, _f32)


def _tc_first_body(x_ref, w_ref, asv_ref, adv_ref,
                   h_ref, as_ref, ad_ref, amax_ref):
    h = jnp.dot(x_ref[...], w_ref[...], preferred_element_type=_f32)
    _tc_common(h, asv_ref[...], adv_ref[...], h_ref, as_ref, ad_ref, amax_ref)


def _tc_next_body(o_ref, d_ref, b_ref, w_ref, asv_ref, adv_ref,
                  h_ref, as_ref, ad_ref, amax_ref):
    den = d_ref[0, :] + d_ref[1, :] + 1e-16
    xin = (o_ref[0] + o_ref[1]) / den[:, None] + b_ref[...][None, :]
    xin = _lr(xin, 0.01)
    h = jnp.dot(xin, w_ref[...], preferred_element_type=_f32)
    _tc_common(h, asv_ref[...], adv_ref[...], h_ref, as_ref, ad_ref, amax_ref)


_TC_OUT = (
    jax.ShapeDtypeStruct((NP, D), _f32),
    jax.ShapeDtypeStruct((NP,), _f32),
    jax.ShapeDtypeStruct((NP,), _f32),
    jax.ShapeDtypeStruct((16,), _f32),
)


def _tc_first(x_pad, W, a_src, a_dst):
    return pl.pallas_call(_tc_first_body, out_shape=_TC_OUT)(
        x_pad, W, a_src, a_dst)


def _tc_next(o, d, b, W, a_src, a_dst):
    return pl.pallas_call(_tc_next_body, out_shape=_TC_OUT)(
        o, d, b, W, a_src, a_dst)


def _pool_body(o_ref, d_ref, b_ref, batch_ref, out_ref):
    den = d_ref[0, :] + d_ref[1, :] + 1e-16
    h = (o_ref[0] + o_ref[1]) / den[:, None] + b_ref[...][None, :]
    h = h[0:N]
    batch = batch_ref[...]
    seg = lax.broadcasted_iota(_i32, (G, N), 0)
    onehot = (seg == batch[None, :]).astype(_f32)
    sums = jnp.dot(onehot, h, preferred_element_type=_f32)
    counts = jnp.sum(onehot, axis=1)
    out_ref[...] = sums / jnp.maximum(counts, 1.0)[:, None]


def _pool(o, d, b, batch):
    return pl.pallas_call(
        _pool_body,
        out_shape=jax.ShapeDtypeStruct((G, D), _f32),
    )(o, d, b, batch)


_MESH = plsc.VectorSubcoreMesh(
    core_axis_name="c", subcore_axis_name="s", num_cores=NC, num_subcores=NS)


@functools.partial(
    pl.kernel,
    out_type=(
        jax.ShapeDtypeStruct((NC, NP, D), _f32),
        jax.ShapeDtypeStruct((NC, DR, D), _f32),
    ),
    mesh=_MESH,
    compiler_params=pltpu.CompilerParams(
        needs_layout_passes=False, use_tc_tiling_on_sc=False),
    scratch_types=[
        pltpu.VMEM((NT,), _f32),
        pltpu.VMEM((NT,), _f32),
        pltpu.VMEM((DR, D), _f32),
        pltpu.VMEM((16,), _f32),
        pltpu.VMEM((GB, B), _i32),
        pltpu.VMEM((GB, B), _i32),
        pltpu.VMEM((GB * B,), _f32),
        pltpu.VMEM((B, D), _f32),
        pltpu.VMEM((B, D), _f32),
        pltpu.VMEM((DR,), _i32),
        pltpu.VMEM_SHARED((NP, D), _f32),
        pltpu.VMEM_SHARED((DR, D), _f32),
        pltpu.SemaphoreType.DMA,
        pltpu.SemaphoreType.DMA,
        pltpu.SemaphoreType.DMA,
        pltpu.SemaphoreType.DMA,
    ],
)
def _edge_kernel(as_hbm, ad_hbm, am_hbm, src_hbm, dst_hbm, h_hbm,
                 acc_out, den_out,
                 as_v, ad_v, d_v, am_v, sidx, didx, pbuf, rows0, rows1,
                 id_v, acc_sh, den_sh, semg0, semg1, sems0, sems1):
    cid = lax.axis_index("c")
    sid = lax.axis_index("s")
    wid = cid * NS + sid
    gbase0 = wid * (EW // B)

    pltpu.sync_copy(as_hbm.at[pl.ds(0, NT)], as_v)
    pltpu.sync_copy(ad_hbm.at[pl.ds(0, NT)], ad_v)
    pltpu.sync_copy(am_hbm, am_v)

    zero16 = jnp.zeros((16,), _f32)
    amax = am_v[...]

    def _zero_d(i, carry):
        d_v[i // 8, pl.ds((i % 8) * 16, 16)] = zero16
        return carry

    lax.fori_loop(0, DR * D // 16, _zero_d, 0)

    def _zero_rows(i, carry):
        rows0[i // 8, pl.ds((i % 8) * 16, 16)] = zero16
        return carry

    lax.fori_loop(0, B * D // 16, _zero_rows, 0)

    def _fill_id(i, carry):
        id_v[pl.ds(i * 16, 16)] = lax.iota(_i32, 16) + i * 16
        return carry

    lax.fori_loop(0, DR // 16, _fill_id, 0)

    def _zero_acc(t, carry):
        pltpu.sync_copy(rows0, acc_sh.at[pl.ds(sid * RPW + t * B, B)])
        return carry

    lax.fori_loop(0, RPW // B, _zero_acc, 0)
    pltpu.sync_copy(rows0.at[pl.ds(0, DR // NS)],
                    den_sh.at[pl.ds(sid * (DR // NS), DR // NS)])
    plsc.subcore_barrier()

    rows = (rows0, rows1)
    semg = (semg0, semg1)
    sems = (sems0, sems1)

    def _group(g, carry):
        grow = gbase0 + g * GB
        pltpu.sync_copy(src_hbm.at[pl.ds(grow, GB)], sidx)
        pltpu.sync_copy(dst_hbm.at[pl.ds(grow, GB)], didx)

        gat = [None, None]
        gat[0] = pltpu.async_copy(h_hbm.at[sidx.at[0]], rows0, semg0)

        for jj in range(GB):
            for k in range(B // 16):
                bsl = pl.ds(k * 16, 16)
                s16 = sidx[jj, bsl]
                d16 = didx[jj, bsl]
                sv = plsc.load_gather(as_v, [s16])
                dv = plsc.load_gather(ad_v, [d16])
                z = sv + dv
                e = jnp.maximum(z, 0.2 * z)
                zc = amax + dv
                cg = jnp.maximum(zc, 0.2 * zc)
                p16 = jnp.exp(e - cg)
                p16 = jnp.where(d16 < N, p16, 0.0)
                pbuf[pl.ds(jj * B + k * 16, 16)] = p16
                plsc.addupdate_scatter(d_v, [d16 >> 7, d16 & 127], p16)

        sca = [None, None]
        for jj in range(GB):
            bb = jj & 1
            if jj + 1 < GB:
                if sca[1 - bb] is not None:
                    sca[1 - bb].wait()
                gat[1 - bb] = pltpu.async_copy(
                    h_hbm.at[sidx.at[jj + 1]], rows[1 - bb], semg[1 - bb])
            gat[bb].wait()

            @plsc.parallel_loop(0, B, unroll=2)
            def _scale(r, _jj=jj, _bb=bb):
                a16 = plsc.load_gather(
                    pbuf, [jnp.full((16,), _jj * B, _i32) + r])
                rbuf = rows[_bb]
                for k in range(D // 16):
                    sl = pl.ds(k * 16, 16)
                    rbuf[r, sl] = rbuf[r, sl] * a16

            sca[bb] = pltpu.async_copy(
                rows[bb], acc_sh.at[didx.at[jj]], sems[bb], add=True)

        sca[0].wait()
        sca[1].wait()
        return carry

    lax.fori_loop(0, NG, _group, 0)

    pltpu.sync_copy(d_v, den_sh.at[id_v], add=True)
    plsc.subcore_barrier()

    pltpu.sync_copy(den_sh.at[pl.ds(sid * (DR // NS), DR // NS)],
                    den_out.at[cid, pl.ds(sid * (DR // NS), DR // NS)])
    pltpu.sync_copy(acc_sh.at[pl.ds(sid * RPW, RPW)],
                    acc_out.at[cid, pl.ds(sid * RPW, RPW)])


def kernel(x, edge_index, batch, W1, a_src1, a_dst1, b1,
           W2, a_src2, a_dst2, b2, W3, a_src3, a_dst3, b3):
    src = edge_index[0].astype(_i32)
    dst = edge_index[1].astype(_i32)
    ppw = (EP - E) // NW
    pad_src = jnp.arange(NW * ppw, dtype=_i32).reshape(NW, ppw) % N
    pad_dst = N + (jnp.arange(NW * ppw, dtype=_i32).reshape(NW, ppw) % (NT - N))
    srcp = jnp.concatenate(
        [src.reshape(NW, E // NW), pad_src], axis=1).reshape(EP // B, B)
    dstp = jnp.concatenate(
        [dst.reshape(NW, E // NW), pad_dst], axis=1).reshape(EP // B, B)
    x_pad = jnp.concatenate([x, jnp.zeros((NP - N, D), _f32)])
    batch32 = batch.astype(_i32)

    h, a_s, a_d, am = _tc_first(x_pad, W1, a_src1, a_dst1)
    o, d = _edge_kernel(a_s, a_d, am, srcp, dstp, h)
    d = d.reshape(NC, NP)

    h, a_s, a_d, am = _tc_next(o, d, b1, W2, a_src2, a_dst2)
    o, d = _edge_kernel(a_s, a_d, am, srcp, dstp, h)
    d = d.reshape(NC, NP)

    h, a_s, a_d, am = _tc_next(o, d, b2, W3, a_src3, a_dst3)
    o, d = _edge_kernel(a_s, a_d, am, srcp, dstp, h)
    d = d.reshape(NC, NP)

    return _pool(o, d, b3, batch32)
